# baseline (speedup 1.0000x reference)
import functools
import math

import jax
import jax.numpy as jnp
from jax.experimental import pallas as pl
from jax.experimental.pallas import tpu as pltpu
from jax.experimental.pallas import tpu_sc as plsc

_SCALE = 0.1
_BN = 1024
_BK = 1024
_GW = 128
_WC = 72


def _prolog_body(mu_ref, ls_ref, lp_ref, wm_ref):
    mu = mu_ref[...]
    ls = ls_ref[...]
    lp = lp_ref[...]
    d = mu.shape[1]
    bk = mu.shape[0]
    iv = jnp.exp(-2.0 * ls)
    miv = mu * iv
    b2 = (-0.5 * jnp.sum(mu * mu * iv, axis=1, keepdims=True)
          - jnp.sum(ls, axis=1, keepdims=True)
          - 0.5 * d * math.log(2.0 * math.pi) + lp)
    h1 = b2.astype(jnp.bfloat16)
    r1 = b2 - h1.astype(jnp.float32)
    h2 = r1.astype(jnp.bfloat16)
    h3 = (r1 - h2.astype(jnp.float32)).astype(jnp.bfloat16)
    zpad = jnp.zeros((bk, _WC - 2 * d - 3), jnp.bfloat16)
    wm_ref[...] = jnp.concatenate(
        [(-0.5 * iv).astype(jnp.bfloat16), miv.astype(jnp.bfloat16),
         h1, h2, h3, zpad], axis=1)


def _prolog(means, log_stds, logpz_col):
    kk, d = means.shape
    return pl.pallas_call(
        _prolog_body,
        grid=(kk // _BK,),
        in_specs=[
            pl.BlockSpec((_BK, d), lambda j: (j, 0)),
            pl.BlockSpec((_BK, d), lambda j: (j, 0)),
            pl.BlockSpec((_BK, 1), lambda j: (j, 0)),
        ],
        out_specs=pl.BlockSpec((_BK, _WC), lambda j: (j, 0)),
        out_shape=jax.ShapeDtypeStruct((kk, _WC), jnp.bfloat16),
    )(means, log_stds, logpz_col)


def _stats_body(inv_nk, x_ref, wm_ref, idx_ref, q_ref,
                m_sc, se_sc, sv_sc, bi_sc, qa_sc):
    k = pl.program_id(1)
    nk = pl.num_programs(1)
    n = pl.program_id(0)
    bk = _BK

    @pl.when(k == 0)
    def _():
        m_sc[...] = jnp.full(m_sc.shape, -jnp.inf, jnp.float32)
        se_sc[...] = jnp.zeros(se_sc.shape, jnp.float32)
        sv_sc[...] = jnp.zeros(sv_sc.shape, jnp.float32)
        bi_sc[...] = jnp.zeros(bi_sc.shape, jnp.int32)

    @pl.when((k == 0) & (n == 0))
    def _():
        qa_sc[...] = jnp.zeros(qa_sc.shape, jnp.float32)

    x = x_ref[...]
    d = x.shape[1]
    bn = x.shape[0]
    amat = jnp.concatenate(
        [(x * x).astype(jnp.bfloat16), x.astype(jnp.bfloat16),
         jnp.ones((bn, 3), jnp.bfloat16),
         jnp.zeros((bn, _WC - 2 * d - 3), jnp.bfloat16)], axis=1)
    l = jax.lax.dot_general(
        wm_ref[...], amat, (((1,), (1,)), ((), ())),
        preferred_element_type=jnp.float32)

    m_old = m_sc[...]
    bmax = jnp.max(l, axis=0, keepdims=True)
    barg0 = jnp.argmax(l, axis=0)[None].astype(jnp.int32)
    e = jnp.exp(l)
    ones_row = jnp.ones((1, bk), jnp.bfloat16)
    e_bf = e.astype(jnp.bfloat16)
    p_bf = (e * l).astype(jnp.bfloat16)
    se_b = jax.lax.dot_general(
        ones_row, e_bf, (((1,), (0,)), ((), ())),
        preferred_element_type=jnp.float32)
    sv_b = jax.lax.dot_general(
        ones_row, p_bf, (((1,), (0,)), ((), ())),
        preferred_element_type=jnp.float32)
    se_new = se_sc[...] + se_b
    sv_new = sv_sc[...] + sv_b

    bi_new = jnp.where(bmax > m_old, barg0 + k * bk, bi_sc[...])

    m_sc[...] = jnp.maximum(m_old, bmax)
    se_sc[...] = se_new
    sv_sc[...] = sv_new
    bi_sc[...] = bi_new

    @pl.when(k == nk - 1)
    def _():
        idx_ref[...] = bi_new[None]
        qa_sc[...] = qa_sc[...] + jnp.sum(sv_new / se_new, keepdims=True)
        q_ref[...] = qa_sc[...] * inv_nk


def _gmm_stats(x, wm, kk):
    n, d = x.shape
    nb = n // _BN
    return pl.pallas_call(
        functools.partial(_stats_body, 1.0 / (float(n) * float(kk))),
        grid=(nb, kk // _BK),
        in_specs=[
            pl.BlockSpec((_BN, d), lambda i, j: (i, 0)),
            pl.BlockSpec((_BK, _WC), lambda i, j: (j, 0)),
        ],
        out_specs=[
            pl.BlockSpec((1, 1, _BN), lambda i, j: (i, 0, 0)),
            pl.BlockSpec((1, 1), lambda i, j: (0, 0)),
        ],
        out_shape=[
            jax.ShapeDtypeStruct((nb, 1, _BN), jnp.int32),
            jax.ShapeDtypeStruct((1, 1), jnp.float32),
        ],
        scratch_shapes=[
            pltpu.VMEM((1, _BN), jnp.float32),
            pltpu.VMEM((1, _BN), jnp.float32),
            pltpu.VMEM((1, _BN), jnp.float32),
            pltpu.VMEM((1, _BN), jnp.int32),
            pltpu.VMEM((1, 1), jnp.float32),
        ],
        compiler_params=pltpu.CompilerParams(
            dimension_semantics=("arbitrary", "arbitrary")),
    )(x, wm)


def _sc_gather(table, idx_row):
    n = idx_row.shape[1]
    d = table.shape[1]
    mesh = plsc.VectorSubcoreMesh(core_axis_name="c", subcore_axis_name="s")

    @pl.kernel(
        out_type=jax.ShapeDtypeStruct((n, d), table.dtype),
        mesh=mesh)
    def gather_kernel(tab_hbm, i_hbm, o_hbm):
        def body(i_vmem, o_vmem):
            pltpu.sync_copy(tab_hbm.at[i_vmem.at[0]], o_vmem)

        pltpu.emit_pipeline(
            body,
            grid=(n // _GW,),
            in_specs=[pl.BlockSpec((1, _GW), index_map=lambda i: (0, i))],
            out_specs=[pl.BlockSpec((_GW, d), index_map=lambda i: (i, 0))],
            core_axis_name="s",
            dimension_semantics=(pltpu.PARALLEL,),
        )(i_hbm, o_hbm)

    return gather_kernel(table, idx_row)


def _resample_body(d, g_ref, nz_ref, out_ref):
    mg = g_ref[:, :d]
    lsg = g_ref[:, d:2 * d]
    out_ref[...] = mg + jnp.exp(lsg) * nz_ref[...]


def _resample(g, noise_scaled):
    n, d = noise_scaled.shape
    return pl.pallas_call(
        functools.partial(_resample_body, d),
        out_shape=jax.ShapeDtypeStruct((n, d), jnp.float32),
    )(g, noise_scaled)


def kernel(x, means, log_stds, weights):
    n, d = x.shape
    kk = means.shape[0]
    logpz_col = jax.nn.log_softmax(weights).reshape(kk, 1)
    wm = _prolog(means, log_stds, logpz_col)
    idx3, q = _gmm_stats(x, wm, kk)
    table = jnp.concatenate(
        [means, log_stds, jnp.zeros((kk, 128 - 2 * d), means.dtype)], axis=1)
    noise_scaled = jax.random.normal(
        jax.random.key(1), x.shape, x.dtype) * _SCALE
    g = _sc_gather(table, idx3.reshape(1, n))
    resampled = _resample(g, noise_scaled)
    return resampled, idx3.reshape(n), q.reshape(())


# speedup vs baseline: 1.0194x; 1.0194x over previous
import functools
import math

import jax
import jax.numpy as jnp
from jax.experimental import pallas as pl
from jax.experimental.pallas import tpu as pltpu
from jax.experimental.pallas import tpu_sc as plsc

_SCALE = 0.1
_BN = 1024
_BK = 1024
_GW = 128
_WC = 72


def _prolog_body(mu_ref, ls_ref, lp_ref, wm_ref, tab_ref):
    mu = mu_ref[...]
    ls = ls_ref[...]
    lp = lp_ref[...]
    d = mu.shape[1]
    bk = mu.shape[0]
    tab_ref[...] = jnp.concatenate(
        [mu, ls, jnp.zeros((bk, 128 - 2 * d), jnp.float32)], axis=1)
    iv = jnp.exp(-2.0 * ls)
    miv = mu * iv
    b2 = (-0.5 * jnp.sum(mu * mu * iv, axis=1, keepdims=True)
          - jnp.sum(ls, axis=1, keepdims=True)
          - 0.5 * d * math.log(2.0 * math.pi) + lp)
    h1 = b2.astype(jnp.bfloat16)
    r1 = b2 - h1.astype(jnp.float32)
    h2 = r1.astype(jnp.bfloat16)
    h3 = (r1 - h2.astype(jnp.float32)).astype(jnp.bfloat16)
    zpad = jnp.zeros((bk, _WC - 2 * d - 3), jnp.bfloat16)
    wm_ref[...] = jnp.concatenate(
        [(-0.5 * iv).astype(jnp.bfloat16), miv.astype(jnp.bfloat16),
         h1, h2, h3, zpad], axis=1)


def _prolog(means, log_stds, logpz_col):
    kk, d = means.shape
    return pl.pallas_call(
        _prolog_body,
        grid=(kk // _BK,),
        in_specs=[
            pl.BlockSpec((_BK, d), lambda j: (j, 0)),
            pl.BlockSpec((_BK, d), lambda j: (j, 0)),
            pl.BlockSpec((_BK, 1), lambda j: (j, 0)),
        ],
        out_specs=[
            pl.BlockSpec((_BK, _WC), lambda j: (j, 0)),
            pl.BlockSpec((_BK, 128), lambda j: (j, 0)),
        ],
        out_shape=[
            jax.ShapeDtypeStruct((kk, _WC), jnp.bfloat16),
            jax.ShapeDtypeStruct((kk, 128), jnp.float32),
        ],
    )(means, log_stds, logpz_col)


def _stats_body(inv_nk, x_ref, wm_ref, idx_ref, q_ref,
                m_sc, se_sc, sv_sc, bi_sc, qa_sc):
    k = pl.program_id(1)
    nk = pl.num_programs(1)
    n = pl.program_id(0)
    bk = _BK

    @pl.when(k == 0)
    def _():
        m_sc[...] = jnp.full(m_sc.shape, -jnp.inf, jnp.float32)
        se_sc[...] = jnp.zeros(se_sc.shape, jnp.float32)
        sv_sc[...] = jnp.zeros(sv_sc.shape, jnp.float32)
        bi_sc[...] = jnp.zeros(bi_sc.shape, jnp.int32)

    @pl.when((k == 0) & (n == 0))
    def _():
        qa_sc[...] = jnp.zeros(qa_sc.shape, jnp.float32)

    x = x_ref[...]
    d = x.shape[1]
    bn = x.shape[0]
    amat = jnp.concatenate(
        [(x * x).astype(jnp.bfloat16), x.astype(jnp.bfloat16),
         jnp.ones((bn, 3), jnp.bfloat16),
         jnp.zeros((bn, _WC - 2 * d - 3), jnp.bfloat16)], axis=1)
    l = jax.lax.dot_general(
        wm_ref[...], amat, (((1,), (1,)), ((), ())),
        preferred_element_type=jnp.float32)

    m_old = m_sc[...]
    bmax = jnp.max(l, axis=0, keepdims=True)
    barg0 = jnp.argmax(l, axis=0)[None].astype(jnp.int32)
    e = jnp.exp(l)
    ones_row = jnp.ones((1, bk), jnp.bfloat16)
    e_bf = e.astype(jnp.bfloat16)
    p_bf = (e * l).astype(jnp.bfloat16)
    se_b = jax.lax.dot_general(
        ones_row, e_bf, (((1,), (0,)), ((), ())),
        preferred_element_type=jnp.float32)
    sv_b = jax.lax.dot_general(
        ones_row, p_bf, (((1,), (0,)), ((), ())),
        preferred_element_type=jnp.float32)
    se_new = se_sc[...] + se_b
    sv_new = sv_sc[...] + sv_b

    bi_new = jnp.where(bmax > m_old, barg0 + k * bk, bi_sc[...])

    m_sc[...] = jnp.maximum(m_old, bmax)
    se_sc[...] = se_new
    sv_sc[...] = sv_new
    bi_sc[...] = bi_new

    @pl.when(k == nk - 1)
    def _():
        idx_ref[...] = bi_new[None]
        qa_sc[...] = qa_sc[...] + jnp.sum(sv_new / se_new, keepdims=True)
        q_ref[...] = qa_sc[...] * inv_nk


def _gmm_stats(x, wm, kk):
    n, d = x.shape
    nb = n // _BN
    return pl.pallas_call(
        functools.partial(_stats_body, 1.0 / (float(n) * float(kk))),
        grid=(nb, kk // _BK),
        in_specs=[
            pl.BlockSpec((_BN, d), lambda i, j: (i, 0)),
            pl.BlockSpec((_BK, _WC), lambda i, j: (j, 0)),
        ],
        out_specs=[
            pl.BlockSpec((1, 1, _BN), lambda i, j: (i, 0, 0)),
            pl.BlockSpec((1, 1), lambda i, j: (0, 0)),
        ],
        out_shape=[
            jax.ShapeDtypeStruct((nb, 1, _BN), jnp.int32),
            jax.ShapeDtypeStruct((1, 1), jnp.float32),
        ],
        scratch_shapes=[
            pltpu.VMEM((1, _BN), jnp.float32),
            pltpu.VMEM((1, _BN), jnp.float32),
            pltpu.VMEM((1, _BN), jnp.float32),
            pltpu.VMEM((1, _BN), jnp.int32),
            pltpu.VMEM((1, 1), jnp.float32),
        ],
        compiler_params=pltpu.CompilerParams(
            dimension_semantics=("arbitrary", "arbitrary")),
    )(x, wm)


def _sc_gather(table, idx_row):
    n = idx_row.shape[1]
    d = table.shape[1]
    mesh = plsc.VectorSubcoreMesh(core_axis_name="c", subcore_axis_name="s")

    @pl.kernel(
        out_type=jax.ShapeDtypeStruct((n, d), table.dtype),
        mesh=mesh)
    def gather_kernel(tab_hbm, i_hbm, o_hbm):
        def body(i_vmem, o_vmem):
            pltpu.sync_copy(tab_hbm.at[i_vmem.at[0]], o_vmem)

        pltpu.emit_pipeline(
            body,
            grid=(n // _GW,),
            in_specs=[pl.BlockSpec((1, _GW), index_map=lambda i: (0, i))],
            out_specs=[pl.BlockSpec((_GW, d), index_map=lambda i: (i, 0))],
            core_axis_name="s",
            dimension_semantics=(pltpu.PARALLEL,),
        )(i_hbm, o_hbm)

    return gather_kernel(table, idx_row)


def _resample_body(d, g_ref, nz_ref, out_ref):
    mg = g_ref[:, :d]
    lsg = g_ref[:, d:2 * d]
    out_ref[...] = mg + jnp.exp(lsg) * nz_ref[...]


def _resample(g, noise_scaled):
    n, d = noise_scaled.shape
    return pl.pallas_call(
        functools.partial(_resample_body, d),
        out_shape=jax.ShapeDtypeStruct((n, d), jnp.float32),
    )(g, noise_scaled)


def kernel(x, means, log_stds, weights):
    n, d = x.shape
    kk = means.shape[0]
    logpz_col = jax.nn.log_softmax(weights).reshape(kk, 1)
    wm, table = _prolog(means, log_stds, logpz_col)
    idx3, q = _gmm_stats(x, wm, kk)
    noise_scaled = jax.random.normal(
        jax.random.key(1), x.shape, x.dtype) * _SCALE
    g = _sc_gather(table, idx3.reshape(1, n))
    resampled = _resample(g, noise_scaled)
    return resampled, idx3.reshape(n), q.reshape(())
